# Initial kernel scaffold; baseline (speedup 1.0000x reference)
import jax
import jax.numpy as jnp
from jax.experimental import pallas as pl


def kernel(idx, joint_info, embedding, edge_idx_map, edge_emb_weight, bond_type):
    raise NotImplementedError("write your pallas kernel here")


# speedup vs baseline: 2.3589x; 2.3589x over previous
import jax
import jax.numpy as jnp
from jax import lax
from jax.experimental import pallas as pl
from jax.experimental.pallas import tpu as pltpu
from jax.experimental.pallas import tpu_sc as plsc

NC = 2
NS = 16
NW = NC * NS
LANES = 16


def _make_sc_call(N, V, MJ, ND, ED, E):
    PER_W = N // NW
    C = 640
    NCHUNK = PER_W // C

    def body(idx_hbm, jp_hbm, bb_hbm, emb_hbm, emapf_hbm, ew_hbm, btf_hbm,
             out_hbm,
             idx_v, jp_v, bb_v, fidx_v, msel_v, eidx_v, embr_v, tail_v,
             btab_v, sem_e, sem_m, sem_w):
        wid = lax.axis_index("s") * NC + lax.axis_index("c")
        lane = lax.iota(jnp.int32, LANES)
        pltpu.sync_copy(btf_hbm, btab_v)

        def do_chunk(ch, carry):
            base = wid * PER_W + ch * C
            pltpu.sync_copy(idx_hbm.at[pl.ds(base, C)], idx_v)
            pltpu.sync_copy(jp_hbm.at[pl.ds(base, C)], jp_v)
            pltpu.sync_copy(bb_hbm.at[pl.ds(base, C)], bb_v)
            cp_emb = pltpu.async_copy(emb_hbm.at[idx_v], embr_v, sem_e)

            def fidx_body(i, c2):
                s = pl.ds(i * LANES, LANES)
                fidx_v[s] = idx_v[s] * MJ + jp_v[s]
                return c2

            lax.fori_loop(0, C // LANES, fidx_body, 0)
            pltpu.async_copy(emapf_hbm.at[fidx_v], msel_v, sem_m).wait()

            def eidx_body(i, c2):
                s = pl.ds(i * LANES, LANES)
                eidx_v[s] = msel_v[s] + 1
                return c2

            lax.fori_loop(0, C // LANES, eidx_body, 0)
            cp_ee = pltpu.async_copy(ew_hbm.at[eidx_v], tail_v, sem_w)
            cp_emb.wait()
            pltpu.sync_copy(embr_v, out_hbm.at[pl.ds(base, C), pl.ds(0, ND)])
            cp_ee.wait()

            def bond_body(i, c2):
                t16 = lane + i * LANES
                bb16 = bb_v[pl.ds(i * LANES, LANES)]
                for j in range(3):
                    v = plsc.load_gather(btab_v, [bb16 * 3 + j])
                    plsc.store_scatter(
                        tail_v, [t16, jnp.full((LANES,), ED - 3 + j, jnp.int32)], v)
                return c2

            lax.fori_loop(0, C // LANES, bond_body, 0)
            pltpu.sync_copy(tail_v, out_hbm.at[pl.ds(base, C), pl.ds(ND, ED)])
            return carry

        lax.fori_loop(0, NCHUNK, do_chunk, 0)

    D = ND + ED
    return pl.kernel(
        body,
        out_type=jax.ShapeDtypeStruct((N, D), jnp.float32),
        mesh=plsc.VectorSubcoreMesh(core_axis_name="c", subcore_axis_name="s",
                                    num_cores=NC, num_subcores=NS),
        compiler_params=pltpu.CompilerParams(use_tc_tiling_on_sc=False,
                                             needs_layout_passes=False),
        scratch_types=[
            pltpu.VMEM((C,), jnp.int32),
            pltpu.VMEM((C,), jnp.int32),
            pltpu.VMEM((C,), jnp.int32),
            pltpu.VMEM((C,), jnp.int32),
            pltpu.VMEM((C,), jnp.int32),
            pltpu.VMEM((C,), jnp.int32),
            pltpu.VMEM((C, ND), jnp.float32),
            pltpu.VMEM((C, ED), jnp.float32),
            pltpu.VMEM((12,), jnp.float32),
            pltpu.SemaphoreType.DMA,
            pltpu.SemaphoreType.DMA,
            pltpu.SemaphoreType.DMA,
        ],
    )


def kernel(idx, joint_info, embedding, edge_idx_map, edge_emb_weight, bond_type):
    B, L = idx.shape
    N = B * L
    V, ND = embedding.shape
    MJ = edge_idx_map.shape[1]
    E, EW = edge_emb_weight.shape
    ED = EW + 3
    idx_f = idx.reshape(N)
    jp_f = joint_info[..., 0].reshape(N)
    bb_f = joint_info[..., 1].reshape(N)
    emap_f = edge_idx_map.reshape(V * MJ)
    ew_pad = jnp.pad(edge_emb_weight, ((0, 0), (0, ED - EW)))
    bt_f = bond_type.reshape(-1)
    out = _make_sc_call(N, V, MJ, ND, ED, E)(
        idx_f, jp_f, bb_f, embedding, emap_f, ew_pad, bt_f)
    return out.reshape(B, L, ND + ED)


# speedup vs baseline: 2.4782x; 1.0506x over previous
import jax
import jax.numpy as jnp
from jax import lax
from jax.experimental import pallas as pl
from jax.experimental.pallas import tpu as pltpu
from jax.experimental.pallas import tpu_sc as plsc

NC = 2
NS = 16
NW = NC * NS
LANES = 16


def _make_sc_call(N, V, MJ, ND, ED, E):
    PER_W = N // NW
    C = 640
    NCHUNK = PER_W // C
    EW = ED - 3

    def body(idx_hbm, jp_hbm, bb_hbm, emb_hbm, emapt_hbm, ew_hbm, btf_hbm,
             out_hbm,
             idx_v, jp_v, bb_v, fidx_v, msel_v, eidx_v, embr_v, tail_v,
             btab_v, sem_e, sem_m, sem_w):
        wid = lax.axis_index("s") * NC + lax.axis_index("c")
        lane = lax.iota(jnp.int32, LANES)
        pltpu.sync_copy(btf_hbm, btab_v)

        def do_chunk(ch, carry):
            base = wid * PER_W + ch * C
            pltpu.sync_copy(idx_hbm.at[pl.ds(base, C)], idx_v)
            pltpu.sync_copy(jp_hbm.at[pl.ds(base, C)], jp_v)
            pltpu.sync_copy(bb_hbm.at[pl.ds(base, C)], bb_v)
            cp_emb = pltpu.async_copy(emb_hbm.at[idx_v], embr_v, sem_e)

            def fidx_body(i, c2):
                s = pl.ds(i * LANES, LANES)
                fidx_v[s] = jp_v[s] * V + idx_v[s]
                return c2

            lax.fori_loop(0, C // LANES, fidx_body, 0)
            pltpu.async_copy(emapt_hbm.at[fidx_v], msel_v, sem_m).wait()

            def eidx_body(i, c2):
                s = pl.ds(i * LANES, LANES)
                eidx_v[s] = msel_v[s] + 1
                return c2

            lax.fori_loop(0, C // LANES, eidx_body, 0)
            cp_ee = pltpu.async_copy(ew_hbm.at[eidx_v], tail_v, sem_w)
            cp_emb.wait()
            pltpu.sync_copy(embr_v, out_hbm.at[pl.ds(base, C), pl.ds(0, ND)])
            cp_ee.wait()

            def tail_body(i, c2):
                t16 = lane + i * LANES
                bb16 = bb_v[pl.ds(i * LANES, LANES)]
                for j in range(3):
                    v = plsc.load_gather(btab_v, [bb16 * 3 + j])
                    plsc.store_scatter(
                        tail_v, [t16, jnp.full((LANES,), EW + j, jnp.int32)], v)
                return c2

            lax.fori_loop(0, C // LANES, tail_body, 0)
            pltpu.sync_copy(tail_v, out_hbm.at[pl.ds(base, C), pl.ds(ND, ED)])
            return carry

        lax.fori_loop(0, NCHUNK, do_chunk, 0)

    D = ND + ED
    return pl.kernel(
        body,
        out_type=jax.ShapeDtypeStruct((N, D), jnp.float32),
        mesh=plsc.VectorSubcoreMesh(core_axis_name="c", subcore_axis_name="s",
                                    num_cores=NC, num_subcores=NS),
        compiler_params=pltpu.CompilerParams(use_tc_tiling_on_sc=False,
                                             needs_layout_passes=False),
        scratch_types=[
            pltpu.VMEM((C,), jnp.int32),
            pltpu.VMEM((C,), jnp.int32),
            pltpu.VMEM((C,), jnp.int32),
            pltpu.VMEM((C,), jnp.int32),
            pltpu.VMEM((C,), jnp.int32),
            pltpu.VMEM((C,), jnp.int32),
            pltpu.VMEM((C, ND), jnp.float32),
            pltpu.VMEM((C, ED), jnp.float32),
            pltpu.VMEM((12,), jnp.float32),
            pltpu.SemaphoreType.DMA,
            pltpu.SemaphoreType.DMA,
            pltpu.SemaphoreType.DMA,
        ],
    )


def kernel(idx, joint_info, embedding, edge_idx_map, edge_emb_weight, bond_type):
    B, L = idx.shape
    N = B * L
    V, ND = embedding.shape
    MJ = edge_idx_map.shape[1]
    E, EW = edge_emb_weight.shape
    ED = EW + 3
    idx_f = idx.reshape(N)
    jp_f = joint_info[..., 0].reshape(N)
    bb_f = joint_info[..., 1].reshape(N)
    emap_t = edge_idx_map.T.reshape(MJ * V)
    ew_pad = jnp.pad(edge_emb_weight, ((0, 0), (0, ED - EW)))
    bt_f = bond_type.reshape(-1)
    out = _make_sc_call(N, V, MJ, ND, ED, E)(
        idx_f, jp_f, bb_f, embedding, emap_t, ew_pad, bt_f)
    return out.reshape(B, L, ND + ED)


# speedup vs baseline: 3.0231x; 1.2199x over previous
import jax
import jax.numpy as jnp
from jax import lax
from jax.experimental import pallas as pl
from jax.experimental.pallas import tpu as pltpu
from jax.experimental.pallas import tpu_sc as plsc

NC = 2
NS = 16
NW = NC * NS
LANES = 16


def _make_sc_call(N, V, MJ, ND, ED, E):
    PER_W = N // NW
    C = 640
    NCHUNK = PER_W // C
    EW = ED - 3

    def body(*refs):
        (idx_hbm, jp_hbm, bb_hbm, emb_hbm, emapt_hbm) = refs[:5]
        ewc_hbm = refs[5:5 + EW]
        btf_hbm, out_hbm = refs[5 + EW], refs[6 + EW]
        (idx_v, jp_v, bb_v, fidx_v, msel_v, eidx_v, embr_v, eet_v, tail_v,
         btab_v, sem_e, sem_m, sem_w) = refs[7 + EW:]
        wid = lax.axis_index("s") * NC + lax.axis_index("c")
        lane = lax.iota(jnp.int32, LANES)
        pltpu.sync_copy(btf_hbm, btab_v)

        def do_chunk(ch, carry):
            base = wid * PER_W + ch * C
            pltpu.sync_copy(idx_hbm.at[pl.ds(base, C)], idx_v)
            pltpu.sync_copy(jp_hbm.at[pl.ds(base, C)], jp_v)
            pltpu.sync_copy(bb_hbm.at[pl.ds(base, C)], bb_v)
            cp_emb = pltpu.async_copy(emb_hbm.at[idx_v], embr_v, sem_e)

            def fidx_body(i, c2):
                s = pl.ds(i * LANES, LANES)
                fidx_v[s] = jp_v[s] * V + idx_v[s]
                return c2

            lax.fori_loop(0, C // LANES, fidx_body, 0)
            pltpu.async_copy(emapt_hbm.at[fidx_v], msel_v, sem_m).wait()

            def eidx_body(i, c2):
                s = pl.ds(i * LANES, LANES)
                eidx_v[s] = msel_v[s] + 1
                return c2

            lax.fori_loop(0, C // LANES, eidx_body, 0)
            cps = [pltpu.async_copy(ewc_hbm[c].at[eidx_v], eet_v.at[c], sem_w)
                   for c in range(EW)]

            def bond_body(i, c2):
                s = pl.ds(i * LANES, LANES)
                bb16 = bb_v[s]
                for j in range(3):
                    eet_v[EW + j, s] = plsc.load_gather(btab_v, [bb16 * 3 + j])
                return c2

            lax.fori_loop(0, C // LANES, bond_body, 0)
            cp_emb.wait()
            pltpu.sync_copy(embr_v, out_hbm.at[pl.ds(base, C), pl.ds(0, ND)])
            for cp in cps:
                cp.wait()

            def tr_body(i, c2):
                t16 = lane + i * LANES
                s = pl.ds(i * LANES, LANES)
                for c in range(ED):
                    plsc.store_scatter(
                        tail_v, [t16, jnp.full((LANES,), c, jnp.int32)],
                        eet_v[c, s])
                return c2

            lax.fori_loop(0, C // LANES, tr_body, 0)
            pltpu.sync_copy(tail_v, out_hbm.at[pl.ds(base, C), pl.ds(ND, ED)])
            return carry

        lax.fori_loop(0, NCHUNK, do_chunk, 0)

    D = ND + ED
    return pl.kernel(
        body,
        out_type=jax.ShapeDtypeStruct((N, D), jnp.float32),
        mesh=plsc.VectorSubcoreMesh(core_axis_name="c", subcore_axis_name="s",
                                    num_cores=NC, num_subcores=NS),
        compiler_params=pltpu.CompilerParams(use_tc_tiling_on_sc=False,
                                             needs_layout_passes=False),
        scratch_types=[
            pltpu.VMEM((C,), jnp.int32),
            pltpu.VMEM((C,), jnp.int32),
            pltpu.VMEM((C,), jnp.int32),
            pltpu.VMEM((C,), jnp.int32),
            pltpu.VMEM((C,), jnp.int32),
            pltpu.VMEM((C,), jnp.int32),
            pltpu.VMEM((C, ND), jnp.float32),
            pltpu.VMEM((ED, C), jnp.float32),
            pltpu.VMEM((C, ED), jnp.float32),
            pltpu.VMEM((12,), jnp.float32),
            pltpu.SemaphoreType.DMA,
            pltpu.SemaphoreType.DMA,
            pltpu.SemaphoreType.DMA,
        ],
    )


def kernel(idx, joint_info, embedding, edge_idx_map, edge_emb_weight, bond_type):
    B, L = idx.shape
    N = B * L
    V, ND = embedding.shape
    MJ = edge_idx_map.shape[1]
    E, EW = edge_emb_weight.shape
    ED = EW + 3
    idx_f = idx.reshape(N)
    jp_f = joint_info[..., 0].reshape(N)
    bb_f = joint_info[..., 1].reshape(N)
    emap_t = edge_idx_map.T.reshape(MJ * V)
    ew_cols = [edge_emb_weight[:, c] for c in range(EW)]
    bt_f = bond_type.reshape(-1)
    out = _make_sc_call(N, V, MJ, ND, ED, E)(
        idx_f, jp_f, bb_f, embedding, emap_t, *ew_cols, bt_f)
    return out.reshape(B, L, ND + ED)
